# baseline (speedup 1.0000x reference)
import functools

import jax
import jax.numpy as jnp
from jax import lax
from jax.experimental import pallas as pl
from jax.experimental.pallas import tpu as pltpu
from jax.experimental.pallas import tpu_sc as plsc

D = 32
L = 16
NC = 2
NS = 16
NW = NC * NS

ROWS = 1000000
TILES = (ROWS + 127) // 128
BLOCK = TILES * 1024


def _gmf_body(user_hbm, item_hbm, utT_hbm, itT_hbm, wb_hbm, bb_hbm, out_hbm,
              uidx, iidx, uoff, ioff, urf, irf, outv, wv, bv, sem_u, sem_i,
              bpw):
    wid = lax.axis_index("s") * NC + lax.axis_index("c")
    base = wid * bpw

    pltpu.sync_copy(user_hbm.at[pl.ds(base, bpw)], uidx)
    pltpu.sync_copy(item_hbm.at[pl.ds(base, bpw)], iidx)
    pltpu.sync_copy(wb_hbm, wv)
    pltpu.sync_copy(bb_hbm, bv)

    cvecs = [jnp.full((L,), d * ROWS, jnp.int32) for d in range(D)]

    def offsets(g, _):
        j0 = g * L
        rv_u = uidx[pl.ds(j0, L)]
        rv_i = iidx[pl.ds(j0, L)]
        for d in range(D):
            uoff[pl.ds(d * bpw + j0, L)] = rv_u + cvecs[d]
            ioff[pl.ds(d * bpw + j0, L)] = rv_i + cvecs[d]
        return 0

    lax.fori_loop(0, bpw // L, offsets, 0)

    cp_u = pltpu.async_copy(utT_hbm.at[uoff], urf, sem_u)
    cp_i = pltpu.async_copy(itT_hbm.at[ioff], irf, sem_i)
    cp_u.wait()
    cp_i.wait()

    wvecs = [wv[pl.ds(d * L, L)] for d in range(D)]
    bvec = bv[...]

    def group(g, _):
        j0 = g * L
        acc = bvec
        for d in range(D):
            uc = urf[pl.ds(d * bpw + j0, L)]
            ic = irf[pl.ds(d * bpw + j0, L)]
            acc = acc + uc * ic * wvecs[d]
        outv[pl.ds(j0, L)] = acc
        return 0

    lax.fori_loop(0, bpw // L, group, 0)

    pltpu.sync_copy(outv, out_hbm.at[pl.ds(base, bpw)])


def kernel(user, item, user_table, item_table, W, b):
    batch = user.shape[0]
    bpw = batch // NW
    mesh = plsc.VectorSubcoreMesh(core_axis_name="c", subcore_axis_name="s")

    wb = jnp.broadcast_to(W.reshape(D, 1), (D, L)).reshape(D * L)
    wb = wb.astype(jnp.float32)
    bb = jnp.broadcast_to(b.reshape(1), (L,)).astype(jnp.float32)

    k = functools.partial(
        pl.kernel,
        mesh=mesh,
        out_type=jax.ShapeDtypeStruct((batch,), jnp.float32),
        scratch_types=[
            pltpu.VMEM((bpw,), jnp.int32),
            pltpu.VMEM((bpw,), jnp.int32),
            pltpu.VMEM((D * bpw,), jnp.int32),
            pltpu.VMEM((D * bpw,), jnp.int32),
            pltpu.VMEM((D * bpw,), jnp.float32),
            pltpu.VMEM((D * bpw,), jnp.float32),
            pltpu.VMEM((bpw,), jnp.float32),
            pltpu.VMEM((D * L,), jnp.float32),
            pltpu.VMEM((L,), jnp.float32),
            pltpu.SemaphoreType.DMA,
            pltpu.SemaphoreType.DMA,
        ],
        compiler_params=pltpu.CompilerParams(
            needs_layout_passes=False, use_tc_tiling_on_sc=False,
            disable_bounds_checks=True),
    )(functools.partial(_gmf_body, bpw=bpw))

    return k(user.astype(jnp.int32), item.astype(jnp.int32),
             user_table.T.reshape(-1), item_table.T.reshape(-1), wb, bb)


# speedup vs baseline: 5.5812x; 5.5812x over previous
import functools

import jax
import jax.numpy as jnp
from jax import lax
from jax.experimental import pallas as pl
from jax.experimental.pallas import tpu as pltpu
from jax.experimental.pallas import tpu_sc as plsc

D = 32
L = 16
NC = 2
NS = 16
NW = NC * NS
PK = 4
PW = PK * D
HB = 2


def _gmf_body(user_hbm, item_hbm, ut_hbm, it_hbm, wb_hbm, bb_hbm, out_hbm,
              uidx, iidx, ublk, iblk, urb, irb, outv, wv, bv, sem_u, sem_i,
              bpw):
    wid = lax.axis_index("s") * NC + lax.axis_index("c")
    base = wid * bpw
    ch = bpw // HB

    pltpu.sync_copy(user_hbm.at[pl.ds(base, bpw)], uidx)
    pltpu.sync_copy(item_hbm.at[pl.ds(base, bpw)], iidx)
    pltpu.sync_copy(wb_hbm, wv)
    pltpu.sync_copy(bb_hbm, bv)

    def blkidx(g, _):
        j0 = g * L
        ublk[pl.ds(j0, L)] = uidx[pl.ds(j0, L)] >> 2
        iblk[pl.ds(j0, L)] = iidx[pl.ds(j0, L)] >> 2
        return 0

    lax.fori_loop(0, bpw // L, blkidx, 0)

    wvecs = [wv[pl.ds(d * L, L)] for d in range(D)]
    bvec = bv[...]
    lane = lax.iota(jnp.int32, L)
    subm = jnp.full((L,), PK - 1, jnp.int32)
    d32 = jnp.full((L,), D, jnp.int32)

    for h in range(HB):
        cp_u = pltpu.async_copy(
            ut_hbm.at[ublk.at[pl.ds(h * ch, ch)]], urb, sem_u)
        cp_i = pltpu.async_copy(
            it_hbm.at[iblk.at[pl.ds(h * ch, ch)]], irb, sem_i)
        cp_u.wait()
        cp_i.wait()

        def group(g, _):
            j0 = h * ch + g * L
            rows = g * L + lane
            cu = (uidx[pl.ds(j0, L)] & subm) * d32
            ci = (iidx[pl.ds(j0, L)] & subm) * d32
            acc = bvec
            for d in range(D):
                cd = jnp.full((L,), d, jnp.int32)
                uc = plsc.load_gather(urb, [rows, cu + cd])
                ic = plsc.load_gather(irb, [rows, ci + cd])
                acc = acc + uc * ic * wvecs[d]
            outv[pl.ds(j0, L)] = acc
            return 0

        lax.fori_loop(0, ch // L, group, 0)

    pltpu.sync_copy(outv, out_hbm.at[pl.ds(base, bpw)])


def kernel(user, item, user_table, item_table, W, b):
    batch = user.shape[0]
    bpw = batch // NW
    ch = bpw // HB
    mesh = plsc.VectorSubcoreMesh(core_axis_name="c", subcore_axis_name="s")

    wb = jnp.broadcast_to(W.reshape(D, 1), (D, L)).reshape(D * L)
    wb = wb.astype(jnp.float32)
    bb = jnp.broadcast_to(b.reshape(1), (L,)).astype(jnp.float32)

    ut4 = user_table.reshape(user_table.shape[0] // PK, PW)
    it4 = item_table.reshape(item_table.shape[0] // PK, PW)

    k = functools.partial(
        pl.kernel,
        mesh=mesh,
        out_type=jax.ShapeDtypeStruct((batch,), jnp.float32),
        scratch_types=[
            pltpu.VMEM((bpw,), jnp.int32),
            pltpu.VMEM((bpw,), jnp.int32),
            pltpu.VMEM((bpw,), jnp.int32),
            pltpu.VMEM((bpw,), jnp.int32),
            pltpu.VMEM((ch, PW), jnp.float32),
            pltpu.VMEM((ch, PW), jnp.float32),
            pltpu.VMEM((bpw,), jnp.float32),
            pltpu.VMEM((D * L,), jnp.float32),
            pltpu.VMEM((L,), jnp.float32),
            pltpu.SemaphoreType.DMA,
            pltpu.SemaphoreType.DMA,
        ],
        compiler_params=pltpu.CompilerParams(
            needs_layout_passes=False, use_tc_tiling_on_sc=False),
    )(functools.partial(_gmf_body, bpw=bpw))

    return k(user.astype(jnp.int32), item.astype(jnp.int32),
             ut4, it4, wb, bb)


# speedup vs baseline: 7.7303x; 1.3851x over previous
import functools

import jax
import jax.numpy as jnp
from jax import lax
from jax.experimental import pallas as pl
from jax.experimental.pallas import tpu as pltpu
from jax.experimental.pallas import tpu_sc as plsc

D = 32
L = 16
NC = 2
NS = 16
NW = NC * NS
ROWS = 1000000
PK = 4
PW = PK * D
HB = 2

SCW = 4096
SRW = SCW // PK


def _stage_body(in_ref, out_ref, xs_ref):
    xs_ref[...] = in_ref[...].T
    for s in range(PK):
        out_ref[:, s * D:(s + 1) * D] = xs_ref[s::PK, :]


def _stage(tableT):
    return pl.pallas_call(
        _stage_body,
        grid=(pl.cdiv(ROWS, SCW),),
        in_specs=[pl.BlockSpec((D, SCW), lambda p: (0, p))],
        out_specs=pl.BlockSpec((SRW, PW), lambda p: (p, 0)),
        out_shape=jax.ShapeDtypeStruct((ROWS // PK, PW), jnp.float32),
        scratch_shapes=[pltpu.VMEM((SCW, D), jnp.float32)],
    )(tableT)


def _gmf_body(user_hbm, item_hbm, ut_hbm, it_hbm, wb_hbm, bb_hbm, out_hbm,
              uidx, iidx, ublk, iblk, urb, irb, outv, wv, bv, sem_u, sem_i,
              bpw):
    wid = lax.axis_index("s") * NC + lax.axis_index("c")
    base = wid * bpw
    ch = bpw // HB

    pltpu.sync_copy(user_hbm.at[pl.ds(base, bpw)], uidx)
    pltpu.sync_copy(item_hbm.at[pl.ds(base, bpw)], iidx)
    pltpu.sync_copy(wb_hbm, wv)
    pltpu.sync_copy(bb_hbm, bv)

    def blkidx(g, _):
        j0 = g * L
        ublk[pl.ds(j0, L)] = uidx[pl.ds(j0, L)] >> 2
        iblk[pl.ds(j0, L)] = iidx[pl.ds(j0, L)] >> 2
        return 0

    lax.fori_loop(0, bpw // L, blkidx, 0)

    wvecs = [wv[pl.ds(d * L, L)] for d in range(D)]
    bvec = bv[...]
    lane = lax.iota(jnp.int32, L)
    subm = jnp.full((L,), PK - 1, jnp.int32)
    d32 = jnp.full((L,), D, jnp.int32)

    for h in range(HB):
        cp_u = pltpu.async_copy(
            ut_hbm.at[ublk.at[pl.ds(h * ch, ch)]], urb, sem_u)
        cp_i = pltpu.async_copy(
            it_hbm.at[iblk.at[pl.ds(h * ch, ch)]], irb, sem_i)
        cp_u.wait()
        cp_i.wait()

        def group(g, _):
            j0 = h * ch + g * L
            rows = g * L + lane
            cu = (uidx[pl.ds(j0, L)] & subm) * d32
            ci = (iidx[pl.ds(j0, L)] & subm) * d32
            acc = bvec
            for d in range(D):
                cd = jnp.full((L,), d, jnp.int32)
                uc = plsc.load_gather(urb, [rows, cu + cd])
                ic = plsc.load_gather(irb, [rows, ci + cd])
                acc = acc + uc * ic * wvecs[d]
            outv[pl.ds(j0, L)] = acc
            return 0

        lax.fori_loop(0, ch // L, group, 0)

    pltpu.sync_copy(outv, out_hbm.at[pl.ds(base, bpw)])


def kernel(user, item, user_table, item_table, W, b):
    batch = user.shape[0]
    bpw = batch // NW
    ch = bpw // HB
    mesh = plsc.VectorSubcoreMesh(core_axis_name="c", subcore_axis_name="s")

    wb = jnp.broadcast_to(W.reshape(D, 1), (D, L)).reshape(D * L)
    wb = wb.astype(jnp.float32)
    bb = jnp.broadcast_to(b.reshape(1), (L,)).astype(jnp.float32)

    ut4 = _stage(user_table.T)
    it4 = _stage(item_table.T)

    k = functools.partial(
        pl.kernel,
        mesh=mesh,
        out_type=jax.ShapeDtypeStruct((batch,), jnp.float32),
        scratch_types=[
            pltpu.VMEM((bpw,), jnp.int32),
            pltpu.VMEM((bpw,), jnp.int32),
            pltpu.VMEM((bpw,), jnp.int32),
            pltpu.VMEM((bpw,), jnp.int32),
            pltpu.VMEM((ch, PW), jnp.float32),
            pltpu.VMEM((ch, PW), jnp.float32),
            pltpu.VMEM((bpw,), jnp.float32),
            pltpu.VMEM((D * L,), jnp.float32),
            pltpu.VMEM((L,), jnp.float32),
            pltpu.SemaphoreType.DMA,
            pltpu.SemaphoreType.DMA,
        ],
        compiler_params=pltpu.CompilerParams(
            needs_layout_passes=False, use_tc_tiling_on_sc=False),
    )(functools.partial(_gmf_body, bpw=bpw))

    return k(user.astype(jnp.int32), item.astype(jnp.int32),
             ut4, it4, wb, bb)
